# Initial kernel scaffold; baseline (speedup 1.0000x reference)
import functools

import jax
import jax.numpy as jnp
from jax import lax
from jax.experimental import pallas as pl
from jax.experimental.pallas import tpu as pltpu
from jax.experimental.pallas import tpu_sc as plsc

B = 16384
D = 64
DV, DA, DT = 512, 128, 768

NC, NS = 2, 16
NW = NC * NS
BPW = B // NW
CHUNK = 64
NCHUNK = BPW // CHUNK

_sc_mesh = plsc.VectorSubcoreMesh(core_axis_name="c", subcore_axis_name="s")


def _gather_body(uid_hbm, mid_hbm, ut_hbm, vf_hbm, af_hbm, tf_hbm,
                 uout, vout, aout, tout,
                 uidx, midx, ubuf, vbuf, abuf, tbuf, sem):
    wid = lax.axis_index("s") * NC + lax.axis_index("c")
    base = wid * BPW
    pltpu.sync_copy(uid_hbm.at[pl.ds(base, BPW)], uidx)
    pltpu.sync_copy(mid_hbm.at[pl.ds(base, BPW)], midx)
    for c in range(NCHUNK):
        off = c * CHUNK
        cu = pltpu.async_copy(ut_hbm.at[uidx.at[pl.ds(off, CHUNK)]], ubuf, sem)
        cv = pltpu.async_copy(vf_hbm.at[midx.at[pl.ds(off, CHUNK)]], vbuf, sem)
        ca = pltpu.async_copy(af_hbm.at[midx.at[pl.ds(off, CHUNK)]], abuf, sem)
        ct = pltpu.async_copy(tf_hbm.at[midx.at[pl.ds(off, CHUNK)]], tbuf, sem)
        cu.wait()
        cv.wait()
        ca.wait()
        ct.wait()
        pltpu.sync_copy(ubuf, uout.at[pl.ds(base + off, CHUNK)])
        pltpu.sync_copy(vbuf, vout.at[pl.ds(base + off, CHUNK)])
        pltpu.sync_copy(abuf, aout.at[pl.ds(base + off, CHUNK)])
        pltpu.sync_copy(tbuf, tout.at[pl.ds(base + off, CHUNK)])


_gather = pl.kernel(
    _gather_body,
    out_type=[
        jax.ShapeDtypeStruct((B, D), jnp.float32),
        jax.ShapeDtypeStruct((B, DV), jnp.float32),
        jax.ShapeDtypeStruct((B, DA), jnp.float32),
        jax.ShapeDtypeStruct((B, DT), jnp.float32),
    ],
    mesh=_sc_mesh,
    scratch_types=[
        pltpu.VMEM((BPW,), jnp.int32),
        pltpu.VMEM((BPW,), jnp.int32),
        pltpu.VMEM((CHUNK, D), jnp.float32),
        pltpu.VMEM((CHUNK, DV), jnp.float32),
        pltpu.VMEM((CHUNK, DA), jnp.float32),
        pltpu.VMEM((CHUNK, DT), jnp.float32),
        pltpu.SemaphoreType.DMA,
    ],
)


BT = 512


def _fuse_body(u_ref, v_ref, a_ref, t_ref, wv_ref, wa_ref, wt_ref,
               bmm_ref, wout_ref, bout_ref, o_ref):
    memb = jnp.dot(v_ref[...], wv_ref[...], preferred_element_type=jnp.float32)
    memb += jnp.dot(a_ref[...], wa_ref[...], preferred_element_type=jnp.float32)
    memb += jnp.dot(t_ref[...], wt_ref[...], preferred_element_type=jnp.float32)
    memb += bmm_ref[...]
    mu = jnp.sum(memb * u_ref[...], axis=1, keepdims=True)
    o_ref[...] = jax.nn.sigmoid(mu * wout_ref[0, 0] + bout_ref[0, 0])


def _fuse(uemb, mv, ma, mt, Wv, Wa, Wt, bmm, wout, bout):
    grid = (B // BT,)
    return pl.pallas_call(
        _fuse_body,
        grid=grid,
        in_specs=[
            pl.BlockSpec((BT, D), lambda i: (i, 0)),
            pl.BlockSpec((BT, DV), lambda i: (i, 0)),
            pl.BlockSpec((BT, DA), lambda i: (i, 0)),
            pl.BlockSpec((BT, DT), lambda i: (i, 0)),
            pl.BlockSpec((DV, D), lambda i: (0, 0)),
            pl.BlockSpec((DA, D), lambda i: (0, 0)),
            pl.BlockSpec((DT, D), lambda i: (0, 0)),
            pl.BlockSpec((1, D), lambda i: (0, 0)),
            pl.BlockSpec((1, 1), lambda i: (0, 0)),
            pl.BlockSpec((1, 1), lambda i: (0, 0)),
        ],
        out_specs=pl.BlockSpec((BT, 1), lambda i: (i, 0)),
        out_shape=jax.ShapeDtypeStruct((B, 1), jnp.float32),
    )(uemb, mv, ma, mt, Wv, Wa, Wt, bmm, wout, bout)


def kernel(x, user_table, video_feat, audio_feat, text_feat, W_mm, b_mm, W_out, b_out):
    uid = x[0].astype(jnp.int32)
    mid = x[1].astype(jnp.int32)
    uemb, mv, ma, mt = _gather(uid, mid, user_table, video_feat,
                               audio_feat, text_feat)
    Wv = W_mm[:DV]
    Wa = W_mm[DV:DV + DA]
    Wt = W_mm[DV + DA:]
    return _fuse(uemb, mv, ma, mt, Wv, Wa, Wt,
                 b_mm.reshape(1, D), W_out, b_out.reshape(1, 1))


# speedup vs baseline: 3.9628x; 3.9628x over previous
import functools

import jax
import jax.numpy as jnp
from jax import lax
from jax.experimental import pallas as pl
from jax.experimental.pallas import tpu as pltpu
from jax.experimental.pallas import tpu_sc as plsc

B = 16384
D = 64
DV, DA, DT = 512, 128, 768

NC, NS = 2, 16
NW = NC * NS
BPW = B // NW
CHUNK = 64
NCHUNK = BPW // CHUNK

_sc_mesh = plsc.VectorSubcoreMesh(core_axis_name="c", subcore_axis_name="s")


def _gather_body(uid_hbm, mid_hbm, ut_hbm, vf_hbm, af_hbm, tf_hbm,
                 uout, vout, aout, tout,
                 uidx, midx, ubuf, vbuf, abuf, tbuf, sem):
    wid = lax.axis_index("s") * NC + lax.axis_index("c")
    base = wid * BPW
    pltpu.sync_copy(uid_hbm.at[pl.ds(base, BPW)], uidx)
    pltpu.sync_copy(mid_hbm.at[pl.ds(base, BPW)], midx)
    for c in range(NCHUNK):
        off = c * CHUNK
        cu = pltpu.async_copy(ut_hbm.at[uidx.at[pl.ds(off, CHUNK)]], ubuf, sem)
        cv = pltpu.async_copy(vf_hbm.at[midx.at[pl.ds(off, CHUNK)]], vbuf, sem)
        ca = pltpu.async_copy(af_hbm.at[midx.at[pl.ds(off, CHUNK)]], abuf, sem)
        ct = pltpu.async_copy(tf_hbm.at[midx.at[pl.ds(off, CHUNK)]], tbuf, sem)
        cu.wait()
        cv.wait()
        ca.wait()
        ct.wait()
        pltpu.sync_copy(ubuf, uout.at[pl.ds(base + off, CHUNK)])
        pltpu.sync_copy(vbuf, vout.at[pl.ds(base + off, CHUNK)])
        pltpu.sync_copy(abuf, aout.at[pl.ds(base + off, CHUNK)])
        pltpu.sync_copy(tbuf, tout.at[pl.ds(base + off, CHUNK)])


_gather = pl.kernel(
    _gather_body,
    out_type=[
        jax.ShapeDtypeStruct((B, 2 * D), jnp.float32),
        jax.ShapeDtypeStruct((B, DV), jnp.float32),
        jax.ShapeDtypeStruct((B, DA), jnp.float32),
        jax.ShapeDtypeStruct((B, DT), jnp.float32),
    ],
    mesh=_sc_mesh,
    scratch_types=[
        pltpu.VMEM((BPW,), jnp.int32),
        pltpu.VMEM((BPW,), jnp.int32),
        pltpu.VMEM((CHUNK, 2 * D), jnp.float32),
        pltpu.VMEM((CHUNK, DV), jnp.float32),
        pltpu.VMEM((CHUNK, DA), jnp.float32),
        pltpu.VMEM((CHUNK, DT), jnp.float32),
        pltpu.SemaphoreType.DMA,
    ],
)


BT = 512


def _fuse_body(u_ref, v_ref, a_ref, t_ref, wv_ref, wa_ref, wt_ref,
               bmm_ref, wout_ref, bout_ref, o_ref):
    memb = jnp.dot(v_ref[...], wv_ref[...], preferred_element_type=jnp.float32)
    memb += jnp.dot(a_ref[...], wa_ref[...], preferred_element_type=jnp.float32)
    memb += jnp.dot(t_ref[...], wt_ref[...], preferred_element_type=jnp.float32)
    memb += bmm_ref[...]
    mu = jnp.sum(memb * u_ref[:, :D], axis=1, keepdims=True)
    o_ref[...] = jax.nn.sigmoid(mu * wout_ref[0, 0] + bout_ref[0, 0])


def _fuse(uemb, mv, ma, mt, Wv, Wa, Wt, bmm, wout, bout):
    grid = (B // BT,)
    return pl.pallas_call(
        _fuse_body,
        grid=grid,
        in_specs=[
            pl.BlockSpec((BT, 2 * D), lambda i: (i, 0)),
            pl.BlockSpec((BT, DV), lambda i: (i, 0)),
            pl.BlockSpec((BT, DA), lambda i: (i, 0)),
            pl.BlockSpec((BT, DT), lambda i: (i, 0)),
            pl.BlockSpec((DV, D), lambda i: (0, 0)),
            pl.BlockSpec((DA, D), lambda i: (0, 0)),
            pl.BlockSpec((DT, D), lambda i: (0, 0)),
            pl.BlockSpec((1, D), lambda i: (0, 0)),
            pl.BlockSpec((1, 1), lambda i: (0, 0)),
            pl.BlockSpec((1, 1), lambda i: (0, 0)),
        ],
        out_specs=pl.BlockSpec((BT, 1), lambda i: (i, 0)),
        out_shape=jax.ShapeDtypeStruct((B, 1), jnp.float32),
    )(uemb, mv, ma, mt, Wv, Wa, Wt, bmm, wout, bout)


def kernel(x, user_table, video_feat, audio_feat, text_feat, W_mm, b_mm, W_out, b_out):
    uid = x[0].astype(jnp.int32)
    mid = x[1].astype(jnp.int32)
    ut_pad = jnp.pad(user_table, ((0, 0), (0, D)))
    uemb, mv, ma, mt = _gather(uid, mid, ut_pad, video_feat,
                               audio_feat, text_feat)
    Wv = W_mm[:DV]
    Wa = W_mm[DV:DV + DA]
    Wt = W_mm[DV + DA:]
    return _fuse(uemb, mv, ma, mt, Wv, Wa, Wt,
                 b_mm.reshape(1, D), W_out, b_out.reshape(1, 1))
